# baseline (speedup 1.0000x reference)
import functools

import jax
import jax.numpy as jnp
from jax import lax
from jax.experimental import pallas as pl
from jax.experimental.pallas import tpu as pltpu
from jax.experimental.pallas import tpu_sc as plsc

N = 10000
D = 128
H = 2
E = 160000
RN = 1000
R = 3
NLAYERS = 2

LANES = 16
NTILES = 32
CHUNK = 80
EPAD = 163840
EPT = EPAD // NTILES
NCHUNKS = EPT // CHUNK
ROWS = 10080
CH = 136
NTAB = 10016


def _hal_body(x_ref, w_ref, asrc_ref, adst_ref, h_ref, al_ref):
    x = x_ref[...]
    for h in range(H):
        hh = jnp.dot(x, w_ref[h], preferred_element_type=jnp.float32)
        h_ref[h] = hh
        al_ref[:, h:h + 1] = lax.dot_general(
            hh, asrc_ref[h:h + 1, :], (((1,), (1,)), ((), ())),
            preferred_element_type=jnp.float32)
        al_ref[:, H + h:H + h + 1] = lax.dot_general(
            hh, adst_ref[h:h + 1, :], (((1,), (1,)), ((), ())),
            preferred_element_type=jnp.float32)


def _hidden_al(x, gw, gas, gad):
    bn = 1000
    return pl.pallas_call(
        _hal_body,
        grid=(N // bn,),
        in_specs=[pl.BlockSpec((bn, D), lambda i: (i, 0)),
                  pl.BlockSpec((H, D, D), lambda i: (0, 0, 0)),
                  pl.BlockSpec((H, D), lambda i: (0, 0)),
                  pl.BlockSpec((H, D), lambda i: (0, 0))],
        out_specs=[pl.BlockSpec((H, bn, D), lambda i: (0, i, 0)),
                   pl.BlockSpec((bn, 2 * H), lambda i: (i, 0))],
        out_shape=[jax.ShapeDtypeStruct((H, N, D), jnp.float32),
                   jax.ShapeDtypeStruct((N, 2 * H), jnp.float32)],
    )(x, gw, gas, gad)


def _sc_edge_agg(h0, h1, als0, ald0, als1, ald1, srcp, dstp):
    mesh = plsc.VectorSubcoreMesh(core_axis_name="c", subcore_axis_name="s")

    @functools.partial(
        pl.kernel,
        mesh=mesh,
        out_type=jax.ShapeDtypeStruct((H, 2 * ROWS, CH), jnp.float32),
        compiler_params=pltpu.CompilerParams(needs_layout_passes=False,
                                             use_tc_tiling_on_sc=False),
        scratch_types=[
            pltpu.VMEM_SHARED((ROWS, CH), jnp.float32),
            pltpu.VMEM((CHUNK,), jnp.int32),
            pltpu.VMEM((CHUNK,), jnp.int32),
            pltpu.VMEM((CHUNK,), jnp.float32),
            pltpu.VMEM((CHUNK,), jnp.float32),
            pltpu.VMEM((CHUNK,), jnp.float32),
            pltpu.VMEM((CHUNK, D), jnp.float32),
            pltpu.VMEM((CHUNK, CH), jnp.float32),
            pltpu.SemaphoreType.DMA,
            pltpu.SemaphoreType.DMA,
        ],
    )
    def k(h0_hbm, h1_hbm, als0_hbm, ald0_hbm, als1_hbm, ald1_hbm,
          src_hbm, dst_hbm, out_hbm,
          acc, src_v, dst_v, alv_v, adv_v, w_v, rows_v, stage_v, sem, sem2):
        cid = lax.axis_index("c")
        sid = lax.axis_index("s")
        wid = cid * 16 + sid
        iota = lax.iota(jnp.int32, LANES)
        rps = ROWS // 16

        for hp in range(H):
            h_hbm = (h0_hbm, h1_hbm)[hp]
            as_hbm = (als0_hbm, als1_hbm)[hp]
            ad_hbm = (ald0_hbm, ald1_hbm)[hp]

            def zb(i, _):
                for dpart in range(CH // LANES):
                    stage_v[i, pl.ds(dpart * LANES, LANES)] = (
                        jnp.zeros((LANES,), jnp.float32))
                return 0
            lax.fori_loop(0, CHUNK, zb, 0)

            def zacc(j, _):
                t = sid + j * 16

                @pl.when(t < ROWS // CHUNK)
                def _():
                    pltpu.sync_copy(stage_v, acc.at[pl.ds(t * CHUNK, CHUNK)])
                return 0
            lax.fori_loop(0, (ROWS // CHUNK + 15) // 16, zacc, 0)
            plsc.subcore_barrier()

            def chunk_body(c, _):
                base = wid * EPT + c * CHUNK
                pltpu.sync_copy(src_hbm.at[pl.ds(base, CHUNK)], src_v)
                pltpu.sync_copy(dst_hbm.at[pl.ds(base, CHUNK)], dst_v)
                gat = pltpu.async_copy(h_hbm.at[src_v], rows_v, sem)
                ga = pltpu.async_copy(as_hbm.at[src_v], alv_v, sem2)
                gb = pltpu.async_copy(ad_hbm.at[dst_v], adv_v, sem2)
                ga.wait()
                gb.wait()
                for g in range(CHUNK // LANES):
                    av = alv_v[pl.ds(g * LANES, LANES)]
                    bv = adv_v[pl.ds(g * LANES, LANES)]
                    xv = av + bv
                    w = jnp.exp(jnp.maximum(xv, 0.2 * xv))
                    w_v[pl.ds(g * LANES, LANES)] = w
                    plsc.store_scatter(
                        stage_v,
                        [g * LANES + iota, jnp.full((LANES,), D, jnp.int32)],
                        w)
                gat.wait()

                def scale(g, _):
                    w16 = w_v[pl.ds(g * LANES, LANES)]
                    for j in range(LANES):
                        wi = w16[j]
                        i = g * LANES + j
                        for dpart in range(D // LANES):
                            v = rows_v[i, pl.ds(dpart * LANES, LANES)]
                            stage_v[i, pl.ds(dpart * LANES, LANES)] = v * wi
                    return 0
                lax.fori_loop(0, CHUNK // LANES, scale, 0)
                pltpu.sync_copy(stage_v, acc.at[dst_v], add=True)
                return 0
            lax.fori_loop(0, NCHUNKS, chunk_body, 0)
            plsc.subcore_barrier()

            pltpu.sync_copy(
                acc.at[pl.ds(sid * rps, rps)],
                out_hbm.at[hp, pl.ds(cid * ROWS + sid * rps, rps)])
            plsc.subcore_barrier()

    return k(h0, h1, als0, ald0, als1, ald1, srcp, dstp)


def _comb_body(a_ref, b_ref, o_ref):
    accm = None
    for h in range(H):
        num = a_ref[h, :, :D] + b_ref[h, :, :D]
        den = a_ref[h, :, D:D + 1] + b_ref[h, :, D:D + 1]
        v = num / (den + 1e-16)
        e = jnp.where(v > 0, v, jnp.exp(jnp.minimum(v, 0.0)) - 1.0)
        accm = e if accm is None else accm + e
    m = accm * (1.0 / H)
    nrm = jnp.sqrt(jnp.sum(m * m, axis=1, keepdims=True))
    o_ref[...] = m / (nrm + 1e-12)


def _combine(agg):
    bn = 720
    return pl.pallas_call(
        _comb_body,
        grid=(ROWS // bn,),
        in_specs=[pl.BlockSpec((H, bn, CH), lambda i: (0, i, 0)),
                  pl.BlockSpec((H, bn, CH), lambda i: (0, ROWS // bn + i, 0))],
        out_specs=pl.BlockSpec((bn, D), lambda i: (i, 0)),
        out_shape=jax.ShapeDtypeStruct((ROWS, D), jnp.float32),
    )(agg, agg)


_INV_SQRT_D = 0.08838834764831845


def _mha_body(x0_ref, x1_ref, x2_ref, wq_ref, wk_ref, wv_ref, o_ref):
    xs = [x0_ref[...], x1_ref[...], x2_ref[...]]
    for h in range(H):
        q = [jnp.dot(x, wq_ref[h], preferred_element_type=jnp.float32)
             for x in xs]
        kk = [jnp.dot(x, wk_ref[h], preferred_element_type=jnp.float32)
              for x in xs]
        vv = [jnp.dot(x, wv_ref[h], preferred_element_type=jnp.float32)
              for x in xs]
        osum = None
        for r in range(R):
            att = [jnp.sum(q[r] * kk[s], axis=1, keepdims=True) * _INV_SQRT_D
                   for s in range(R)]
            m = jnp.maximum(jnp.maximum(att[0], att[1]), att[2])
            ee = [jnp.exp(a - m) for a in att]
            den = ee[0] + ee[1] + ee[2]
            o_r = (ee[0] * vv[0] + ee[1] * vv[1] + ee[2] * vv[2]) / den
            osum = o_r if osum is None else osum + o_r
        o_ref[:, h * D:(h + 1) * D] = osum * (1.0 / R)


def _mha(x0, x1, x2, wq, wk, wv):
    bn = 1000
    return pl.pallas_call(
        _mha_body,
        grid=(N // bn,),
        in_specs=[pl.BlockSpec((bn, D), lambda i: (i, 0)),
                  pl.BlockSpec((bn, D), lambda i: (i, 0)),
                  pl.BlockSpec((bn, D), lambda i: (i, 0)),
                  pl.BlockSpec((H, D, D), lambda i: (0, 0, 0)),
                  pl.BlockSpec((H, D, D), lambda i: (0, 0, 0)),
                  pl.BlockSpec((H, D, D), lambda i: (0, 0, 0))],
        out_specs=pl.BlockSpec((bn, H * D), lambda i: (i, 0)),
        out_shape=jax.ShapeDtypeStruct((N, H * D), jnp.float32),
    )(x0, x1, x2, wq, wk, wv)


def _proj_body(adj_ref, emb_ref, fused_ref, w_ref, b_ref, o_ref):
    adj = adj_ref[...]
    rs = jnp.sum(adj, axis=1, keepdims=True)
    ragg = jnp.dot(adj, emb_ref[...],
                   preferred_element_type=jnp.float32) / (rs + 1e-5)
    f = jnp.dot(fused_ref[...], w_ref[:H * D, :],
                preferred_element_type=jnp.float32)
    g = jnp.dot(ragg, w_ref[H * D:, :], preferred_element_type=jnp.float32)
    o_ref[...] = jnp.maximum(f + g + b_ref[...], 0.0)


def _relproj(rel_adj, rel_emb, fused, proj_w, proj_b2):
    bn = 1000
    return pl.pallas_call(
        _proj_body,
        grid=(N // bn,),
        in_specs=[pl.BlockSpec((bn, RN), lambda i: (i, 0)),
                  pl.BlockSpec((RN, D), lambda i: (0, 0)),
                  pl.BlockSpec((bn, H * D), lambda i: (i, 0)),
                  pl.BlockSpec((H * D + D, D), lambda i: (0, 0)),
                  pl.BlockSpec((1, D), lambda i: (0, 0))],
        out_specs=pl.BlockSpec((bn, D), lambda i: (i, 0)),
        out_shape=jax.ShapeDtypeStruct((N, D), jnp.float32),
    )(rel_adj, rel_emb, fused, proj_w, proj_b2)


def _forward(ent, rel_emb, rel_adj, edge, gat_w, gat_asrc, gat_adst,
             wq, wk, wv, proj_w, proj_b2):
    npad = EPAD - E
    srcp = jnp.concatenate(
        [edge[0].astype(jnp.int32),
         jnp.arange(npad, dtype=jnp.int32) % N])
    dstp = jnp.concatenate(
        [edge[1].astype(jnp.int32),
         N + jnp.arange(npad, dtype=jnp.int32) % (ROWS - N)])
    xs = [ent]
    x = ent
    for l in range(NLAYERS):
        hml, al = _hidden_al(x, gat_w[l], gat_asrc[l], gat_adst[l])
        alp = jnp.pad(al, ((0, NTAB - N), (0, 0)))
        agg = _sc_edge_agg(hml[0], hml[1], alp[:, 0], alp[:, 2],
                           alp[:, 1], alp[:, 3], srcp, dstp)
        x = _combine(agg)[:N]
        xs.append(x)
    fused = _mha(xs[0], xs[1], xs[2], wq, wk, wv)
    return _relproj(rel_adj, rel_emb, fused, proj_w, proj_b2)


def kernel(ent_sr, ent_tg, rel_emb_sr, rel_emb_tg, rel_adj_sr, rel_adj_tg,
           gat_W, gat_asrc, gat_adst, Wq, Wk, Wv, proj_W, proj_b,
           edge_sr, edge_tg):
    pb = proj_b.reshape(1, D)
    sr = _forward(ent_sr, rel_emb_sr, rel_adj_sr, edge_sr,
                  gat_W, gat_asrc, gat_adst, Wq, Wk, Wv, proj_W, pb)
    tg = _forward(ent_tg, rel_emb_tg, rel_adj_tg, edge_tg,
                  gat_W, gat_asrc, gat_adst, Wq, Wk, Wv, proj_W, pb)
    return (sr, tg)


# speedup vs baseline: 1.0016x; 1.0016x over previous
import functools

import jax
import jax.numpy as jnp
from jax import lax
from jax.experimental import pallas as pl
from jax.experimental.pallas import tpu as pltpu
from jax.experimental.pallas import tpu_sc as plsc

N = 10000
D = 128
H = 2
E = 160000
RN = 1000
R = 3
NLAYERS = 2

LANES = 16
NTILES = 32
CHUNK = 80
EPAD = 163840
EPT = EPAD // NTILES
NCHUNKS = EPT // CHUNK
ROWS = 10080
CH = 144
NTAB = 10016


def _hal_body(x_ref, w_ref, asrc_ref, adst_ref, h_ref, al_ref):
    x = x_ref[...]
    for h in range(H):
        hh = jnp.dot(x, w_ref[h], preferred_element_type=jnp.float32)
        h_ref[h] = hh
        al_ref[:, h:h + 1] = lax.dot_general(
            hh, asrc_ref[h:h + 1, :], (((1,), (1,)), ((), ())),
            preferred_element_type=jnp.float32)
        al_ref[:, H + h:H + h + 1] = lax.dot_general(
            hh, adst_ref[h:h + 1, :], (((1,), (1,)), ((), ())),
            preferred_element_type=jnp.float32)


def _hidden_al(x, gw, gas, gad):
    bn = 1000
    return pl.pallas_call(
        _hal_body,
        grid=(N // bn,),
        in_specs=[pl.BlockSpec((bn, D), lambda i: (i, 0)),
                  pl.BlockSpec((H, D, D), lambda i: (0, 0, 0)),
                  pl.BlockSpec((H, D), lambda i: (0, 0)),
                  pl.BlockSpec((H, D), lambda i: (0, 0))],
        out_specs=[pl.BlockSpec((H, bn, D), lambda i: (0, i, 0)),
                   pl.BlockSpec((bn, 2 * H), lambda i: (i, 0))],
        out_shape=[jax.ShapeDtypeStruct((H, N, D), jnp.float32),
                   jax.ShapeDtypeStruct((N, 2 * H), jnp.float32)],
    )(x, gw, gas, gad)


def _sc_edge_agg(h0, h1, als0, ald0, als1, ald1, srcp, dstp):
    mesh = plsc.VectorSubcoreMesh(core_axis_name="c", subcore_axis_name="s")

    @functools.partial(
        pl.kernel,
        mesh=mesh,
        out_type=jax.ShapeDtypeStruct((H, 2 * ROWS, CH), jnp.float32),
        compiler_params=pltpu.CompilerParams(needs_layout_passes=False,
                                             use_tc_tiling_on_sc=False),
        scratch_types=[
            pltpu.VMEM_SHARED((ROWS, CH), jnp.float32),
            pltpu.VMEM((CHUNK,), jnp.int32),
            pltpu.VMEM((CHUNK,), jnp.int32),
            pltpu.VMEM((CHUNK,), jnp.float32),
            pltpu.VMEM((CHUNK,), jnp.float32),
            pltpu.VMEM((CHUNK,), jnp.float32),
            pltpu.VMEM((CHUNK, D), jnp.float32),
            pltpu.VMEM((CHUNK, CH), jnp.float32),
            pltpu.SemaphoreType.DMA,
            pltpu.SemaphoreType.DMA,
        ],
    )
    def k(h0_hbm, h1_hbm, als0_hbm, ald0_hbm, als1_hbm, ald1_hbm,
          src_hbm, dst_hbm, out_hbm,
          acc, src_v, dst_v, alv_v, adv_v, w_v, rows_v, stage_v, sem, sem2):
        cid = lax.axis_index("c")
        sid = lax.axis_index("s")
        wid = cid * 16 + sid
        iota = lax.iota(jnp.int32, LANES)
        rps = ROWS // 16

        for hp in range(H):
            h_hbm = (h0_hbm, h1_hbm)[hp]
            as_hbm = (als0_hbm, als1_hbm)[hp]
            ad_hbm = (ald0_hbm, ald1_hbm)[hp]

            def zb(i, _):
                for dpart in range(CH // LANES):
                    stage_v[i, pl.ds(dpart * LANES, LANES)] = (
                        jnp.zeros((LANES,), jnp.float32))
                return 0
            lax.fori_loop(0, CHUNK, zb, 0)

            def zacc(j, _):
                t = sid + j * 16

                @pl.when(t < ROWS // CHUNK)
                def _():
                    pltpu.sync_copy(stage_v, acc.at[pl.ds(t * CHUNK, CHUNK)])
                return 0
            lax.fori_loop(0, (ROWS // CHUNK + 15) // 16, zacc, 0)
            plsc.subcore_barrier()

            def chunk_body(c, _):
                base = wid * EPT + c * CHUNK
                pltpu.sync_copy(src_hbm.at[pl.ds(base, CHUNK)], src_v)
                pltpu.sync_copy(dst_hbm.at[pl.ds(base, CHUNK)], dst_v)
                gat = pltpu.async_copy(h_hbm.at[src_v], rows_v, sem)
                ga = pltpu.async_copy(as_hbm.at[src_v], alv_v, sem2)
                gb = pltpu.async_copy(ad_hbm.at[dst_v], adv_v, sem2)
                ga.wait()
                gb.wait()
                for g in range(CHUNK // LANES):
                    av = alv_v[pl.ds(g * LANES, LANES)]
                    bv = adv_v[pl.ds(g * LANES, LANES)]
                    xv = av + bv
                    w = jnp.exp(jnp.maximum(xv, 0.2 * xv))
                    w_v[pl.ds(g * LANES, LANES)] = w
                    plsc.store_scatter(
                        stage_v,
                        [g * LANES + iota, jnp.full((LANES,), D, jnp.int32)],
                        w)
                gat.wait()

                def scale(g, _):
                    w16 = w_v[pl.ds(g * LANES, LANES)]
                    for j in range(LANES):
                        wi = w16[j]
                        i = g * LANES + j
                        for dpart in range(D // LANES):
                            v = rows_v[i, pl.ds(dpart * LANES, LANES)]
                            stage_v[i, pl.ds(dpart * LANES, LANES)] = v * wi
                    return 0
                lax.fori_loop(0, CHUNK // LANES, scale, 0)
                pltpu.sync_copy(stage_v, acc.at[dst_v], add=True)
                return 0
            lax.fori_loop(0, NCHUNKS, chunk_body, 0)
            plsc.subcore_barrier()

            pltpu.sync_copy(
                acc.at[pl.ds(sid * rps, rps)],
                out_hbm.at[hp, pl.ds(cid * ROWS + sid * rps, rps)])
            plsc.subcore_barrier()

    return k(h0, h1, als0, ald0, als1, ald1, srcp, dstp)


def _comb_body(a_ref, b_ref, o_ref):
    accm = None
    for h in range(H):
        num = a_ref[h, :, :D] + b_ref[h, :, :D]
        den = a_ref[h, :, D:D + 1] + b_ref[h, :, D:D + 1]
        v = num / (den + 1e-16)
        e = jnp.where(v > 0, v, jnp.exp(jnp.minimum(v, 0.0)) - 1.0)
        accm = e if accm is None else accm + e
    m = accm * (1.0 / H)
    nrm = jnp.sqrt(jnp.sum(m * m, axis=1, keepdims=True))
    o_ref[...] = m / (nrm + 1e-12)


def _combine(agg):
    bn = 720
    return pl.pallas_call(
        _comb_body,
        grid=(ROWS // bn,),
        in_specs=[pl.BlockSpec((H, bn, CH), lambda i: (0, i, 0)),
                  pl.BlockSpec((H, bn, CH), lambda i: (0, ROWS // bn + i, 0))],
        out_specs=pl.BlockSpec((bn, D), lambda i: (i, 0)),
        out_shape=jax.ShapeDtypeStruct((ROWS, D), jnp.float32),
    )(agg, agg)


_INV_SQRT_D = 0.08838834764831845


def _mha_body(x0_ref, x1_ref, x2_ref, wq_ref, wk_ref, wv_ref, o_ref):
    xs = [x0_ref[...], x1_ref[...], x2_ref[...]]
    for h in range(H):
        q = [jnp.dot(x, wq_ref[h], preferred_element_type=jnp.float32)
             for x in xs]
        kk = [jnp.dot(x, wk_ref[h], preferred_element_type=jnp.float32)
              for x in xs]
        vv = [jnp.dot(x, wv_ref[h], preferred_element_type=jnp.float32)
              for x in xs]
        osum = None
        for r in range(R):
            att = [jnp.sum(q[r] * kk[s], axis=1, keepdims=True) * _INV_SQRT_D
                   for s in range(R)]
            m = jnp.maximum(jnp.maximum(att[0], att[1]), att[2])
            ee = [jnp.exp(a - m) for a in att]
            den = ee[0] + ee[1] + ee[2]
            o_r = (ee[0] * vv[0] + ee[1] * vv[1] + ee[2] * vv[2]) / den
            osum = o_r if osum is None else osum + o_r
        o_ref[:, h * D:(h + 1) * D] = osum * (1.0 / R)


def _mha(x0, x1, x2, wq, wk, wv):
    bn = 1000
    return pl.pallas_call(
        _mha_body,
        grid=(N // bn,),
        in_specs=[pl.BlockSpec((bn, D), lambda i: (i, 0)),
                  pl.BlockSpec((bn, D), lambda i: (i, 0)),
                  pl.BlockSpec((bn, D), lambda i: (i, 0)),
                  pl.BlockSpec((H, D, D), lambda i: (0, 0, 0)),
                  pl.BlockSpec((H, D, D), lambda i: (0, 0, 0)),
                  pl.BlockSpec((H, D, D), lambda i: (0, 0, 0))],
        out_specs=pl.BlockSpec((bn, H * D), lambda i: (i, 0)),
        out_shape=jax.ShapeDtypeStruct((N, H * D), jnp.float32),
    )(x0, x1, x2, wq, wk, wv)


def _proj_body(adj_ref, emb_ref, fused_ref, w_ref, b_ref, o_ref):
    adj = adj_ref[...]
    rs = jnp.sum(adj, axis=1, keepdims=True)
    ragg = jnp.dot(adj, emb_ref[...],
                   preferred_element_type=jnp.float32) / (rs + 1e-5)
    f = jnp.dot(fused_ref[...], w_ref[:H * D, :],
                preferred_element_type=jnp.float32)
    g = jnp.dot(ragg, w_ref[H * D:, :], preferred_element_type=jnp.float32)
    o_ref[...] = jnp.maximum(f + g + b_ref[...], 0.0)


def _relproj(rel_adj, rel_emb, fused, proj_w, proj_b2):
    bn = 1000
    return pl.pallas_call(
        _proj_body,
        grid=(N // bn,),
        in_specs=[pl.BlockSpec((bn, RN), lambda i: (i, 0)),
                  pl.BlockSpec((RN, D), lambda i: (0, 0)),
                  pl.BlockSpec((bn, H * D), lambda i: (i, 0)),
                  pl.BlockSpec((H * D + D, D), lambda i: (0, 0)),
                  pl.BlockSpec((1, D), lambda i: (0, 0))],
        out_specs=pl.BlockSpec((bn, D), lambda i: (i, 0)),
        out_shape=jax.ShapeDtypeStruct((N, D), jnp.float32),
    )(rel_adj, rel_emb, fused, proj_w, proj_b2)


def _forward(ent, rel_emb, rel_adj, edge, gat_w, gat_asrc, gat_adst,
             wq, wk, wv, proj_w, proj_b2):
    npad = EPAD - E
    srcp = jnp.concatenate(
        [edge[0].astype(jnp.int32),
         jnp.arange(npad, dtype=jnp.int32) % N])
    dstp = jnp.concatenate(
        [edge[1].astype(jnp.int32),
         N + jnp.arange(npad, dtype=jnp.int32) % (ROWS - N)])
    xs = [ent]
    x = ent
    for l in range(NLAYERS):
        hml, al = _hidden_al(x, gat_w[l], gat_asrc[l], gat_adst[l])
        alp = jnp.pad(al, ((0, NTAB - N), (0, 0)))
        agg = _sc_edge_agg(hml[0], hml[1], alp[:, 0], alp[:, 2],
                           alp[:, 1], alp[:, 3], srcp, dstp)
        x = _combine(agg)[:N]
        xs.append(x)
    fused = _mha(xs[0], xs[1], xs[2], wq, wk, wv)
    return _relproj(rel_adj, rel_emb, fused, proj_w, proj_b2)


def kernel(ent_sr, ent_tg, rel_emb_sr, rel_emb_tg, rel_adj_sr, rel_adj_tg,
           gat_W, gat_asrc, gat_adst, Wq, Wk, Wv, proj_W, proj_b,
           edge_sr, edge_tg):
    pb = proj_b.reshape(1, D)
    sr = _forward(ent_sr, rel_emb_sr, rel_adj_sr, edge_sr,
                  gat_W, gat_asrc, gat_adst, Wq, Wk, Wv, proj_W, pb)
    tg = _forward(ent_tg, rel_emb_tg, rel_adj_tg, edge_tg,
                  gat_W, gat_asrc, gat_adst, Wq, Wk, Wv, proj_W, pb)
    return (sr, tg)
